# baseline (speedup 1.0000x reference)
import functools

import jax
import jax.numpy as jnp
from jax import lax
from jax.experimental import pallas as pl
from jax.experimental.pallas import tpu as pltpu
from jax.experimental.pallas import tpu_sc as plsc

_CHUNK = 128
_NBUF = 4
_TBUF = 4


@functools.lru_cache(maxsize=None)
def _make_gather(S, N, V, E):
    info = plsc.get_sparse_core_info()
    nc, ns = info.num_cores, info.num_subcores
    nw = nc * ns
    B = S * N
    assert B % (nw * _CHUNK) == 0 and N % _CHUNK == 0 and E % 8 == 0
    nchunk = B // (nw * _CHUNK)
    nb_per_s = N // _CHUNK
    eb = E // 8
    assert nchunk % _NBUF == 0 and nchunk > _NBUF
    mesh = plsc.VectorSubcoreMesh(core_axis_name="c", subcore_axis_name="s")

    @functools.partial(
        pl.kernel,
        out_type=jax.ShapeDtypeStruct((S, eb, nb_per_s, 8, _CHUNK), jnp.float32),
        mesh=mesh,
        compiler_params=pltpu.CompilerParams(
            use_tc_tiling_on_sc=False, needs_layout_passes=False),
        scratch_types=[
            pltpu.VMEM((nchunk, _CHUNK), jnp.int32),
            pltpu.VMEM((_NBUF, _CHUNK, E), jnp.float32),
            pltpu.VMEM((_TBUF, eb, 8, _CHUNK), jnp.float32),
            pltpu.SemaphoreType.DMA((_NBUF,)),
            pltpu.SemaphoreType.DMA((_TBUF,)),
        ],
    )
    def gather_kernel(idx_hbm, table_hbm, out_hbm, idx_v, rows_v, tr_v,
                      gsem, osem):
        wid = lax.axis_index("s") * nc + lax.axis_index("c")
        c0 = wid * nchunk
        pltpu.sync_copy(idx_hbm.at[wid], idx_v)

        iota = lax.iota(jnp.int32, 16)
        zeros16 = jnp.zeros((16,), jnp.int32)

        def gather_desc(j, b):
            return pltpu.make_async_copy(
                table_hbm.at[idx_v.at[j]], rows_v.at[b], gsem.at[b])

        def ostore_desc(j, b):
            c = c0 + j
            s_ = c >> 7
            nb = c & (nb_per_s - 1)
            return pltpu.make_async_copy(
                tr_v.at[b], out_hbm.at[s_, :, nb], osem.at[b])

        for b in range(_NBUF):
            gather_desc(b, b).start()

        def round_body(r, carry):
            j0 = r * _NBUF
            for b in range(_NBUF):
                j = j0 + b
                gather_desc(j, b).wait()

                @pl.when(r > 0)
                def _():
                    ostore_desc(j - _NBUF, b).wait()

                rows = rows_v.at[b]
                tr = tr_v.at[b]

                def tloop(g, tcarry):
                    row_idx = g * 16 + iota
                    for e in range(E):
                        v = plsc.load_gather(rows, [row_idx, zeros16 + e])
                        tr[e // 8, e % 8, pl.ds(g * 16, 16)] = v
                    return tcarry

                lax.fori_loop(0, _CHUNK // 16, tloop, 0)
                ostore_desc(j, b).start()

                @pl.when(j + _NBUF < nchunk)
                def _():
                    gather_desc(j + _NBUF, b).start()
            return carry

        lax.fori_loop(0, nchunk // _NBUF, round_body, 0)

        j0 = nchunk - _NBUF
        for b in range(_NBUF):
            ostore_desc(j0 + b, b).wait()

    return gather_kernel, nw, nchunk


def kernel(x, table):
    n, s = x.shape
    v, e = table.shape
    fn, nw, nchunk = _make_gather(s, n, v, e)
    idx = x.astype(jnp.int32).T.reshape(nw, nchunk, _CHUNK)
    k6 = fn(idx, table)
    return k6.transpose((2, 4, 0, 1, 3)).reshape(n, s, e)


# speedup vs baseline: 2.4238x; 2.4238x over previous
import functools

import jax
import jax.numpy as jnp
from jax import lax
from jax.experimental import pallas as pl
from jax.experimental.pallas import tpu as pltpu
from jax.experimental.pallas import tpu_sc as plsc

_CHUNK = 128
_NBUF = 4
_TBUF = 4


@functools.lru_cache(maxsize=None)
def _make_gather(S, N, V, E):
    info = plsc.get_sparse_core_info()
    nc, ns = info.num_cores, info.num_subcores
    nw = nc * ns
    B = S * N
    assert B % (nw * _CHUNK) == 0 and N % _CHUNK == 0 and E % 8 == 0
    nchunk = B // (nw * _CHUNK)
    nb_per_s = N // _CHUNK
    eb = E // 8
    assert nchunk % _NBUF == 0 and nchunk > _NBUF
    mesh = plsc.VectorSubcoreMesh(core_axis_name="c", subcore_axis_name="s")

    @functools.partial(
        pl.kernel,
        out_type=jax.ShapeDtypeStruct((S, eb, nb_per_s, 8, _CHUNK), jnp.float32),
        mesh=mesh,
        compiler_params=pltpu.CompilerParams(
            use_tc_tiling_on_sc=False, needs_layout_passes=False),
        scratch_types=[
            pltpu.VMEM((nchunk, _CHUNK), jnp.int32),
            pltpu.VMEM((_NBUF, _CHUNK, E), jnp.float32),
            pltpu.VMEM((_TBUF, eb, 8, _CHUNK + 1), jnp.float32),
            pltpu.SemaphoreType.DMA((_NBUF,)),
            pltpu.SemaphoreType.DMA((_TBUF,)),
        ],
    )
    def gather_kernel(idx_hbm, table_hbm, out_hbm, idx_v, rows_v, tr_v,
                      gsem, osem):
        wid = lax.axis_index("s") * nc + lax.axis_index("c")
        c0 = wid * nchunk
        pltpu.sync_copy(idx_hbm.at[wid], idx_v)

        zeros16 = jnp.zeros((16,), jnp.int32)
        iota = lax.iota(jnp.int32, 16)
        eb_vecs = [(16 * k + iota) // 8 for k in range(E // 16)]
        es_vecs = [(16 * k + iota) % 8 for k in range(E // 16)]

        def gather_desc(j, b):
            return pltpu.make_async_copy(
                table_hbm.at[idx_v.at[j]], rows_v.at[b], gsem.at[b])

        def ostore_desc(j, b):
            c = c0 + j
            s_ = c >> 7
            nb = c & (nb_per_s - 1)
            return pltpu.make_async_copy(
                tr_v.at[b, :, :, pl.ds(0, _CHUNK)],
                out_hbm.at[s_, :, nb], osem.at[b])

        for b in range(_NBUF):
            gather_desc(b, b).start()

        def round_body(r, carry):
            j0 = r * _NBUF
            for b in range(_NBUF):
                j = j0 + b
                gather_desc(j, b).wait()

                @pl.when(r > 0)
                def _():
                    ostore_desc(j - _NBUF, b).wait()

                rows = rows_v.at[b]
                tr = tr_v.at[b]

                @plsc.parallel_loop(0, _CHUNK, unroll=2)
                def _(n):
                    col = zeros16 + n
                    for k in range(E // 16):
                        v = rows[n, pl.ds(16 * k, 16)]
                        plsc.store_scatter(tr, [eb_vecs[k], es_vecs[k], col], v)

                ostore_desc(j, b).start()

                @pl.when(j + _NBUF < nchunk)
                def _():
                    gather_desc(j + _NBUF, b).start()
            return carry

        lax.fori_loop(0, nchunk // _NBUF, round_body, 0)

        j0 = nchunk - _NBUF
        for b in range(_NBUF):
            ostore_desc(j0 + b, b).wait()

    return gather_kernel, nw, nchunk


def kernel(x, table):
    n, s = x.shape
    v, e = table.shape
    fn, nw, nchunk = _make_gather(s, n, v, e)
    idx = x.astype(jnp.int32).T.reshape(nw, nchunk, _CHUNK)
    k6 = fn(idx, table)
    return k6.transpose((2, 4, 0, 1, 3)).reshape(n, s, e)


# speedup vs baseline: 3.6516x; 1.5066x over previous
import functools

import jax
import jax.numpy as jnp
from jax import lax
from jax.experimental import pallas as pl
from jax.experimental.pallas import tpu as pltpu
from jax.experimental.pallas import tpu_sc as plsc

_CHUNK = 128
_NBUF = 4
_TBUF = 4
_HALF_BITS = 19
_HALF = 1 << _HALF_BITS
_VBLK = 2048


def _b0(V):
    return (V - _HALF + _VBLK - 1) // _VBLK * _VBLK


@functools.lru_cache(maxsize=None)
def _make_gather(S, N, V, E):
    info = plsc.get_sparse_core_info()
    nc, ns = info.num_cores, info.num_subcores
    nw = nc * ns
    B = S * N
    assert B % (nw * _CHUNK) == 0 and N % _CHUNK == 0 and E % 8 == 0
    nchunk = B // (nw * _CHUNK)
    nb_per_s = N // _CHUNK
    eb = E // 8
    assert nchunk % _NBUF == 0 and nchunk > _NBUF
    mesh = plsc.VectorSubcoreMesh(core_axis_name="c", subcore_axis_name="s")

    @functools.partial(
        pl.kernel,
        out_type=jax.ShapeDtypeStruct((S, eb, nb_per_s, 8, _CHUNK), jnp.float32),
        mesh=mesh,
        compiler_params=pltpu.CompilerParams(
            use_tc_tiling_on_sc=False, needs_layout_passes=False),
        scratch_types=[
            pltpu.VMEM((nchunk, _CHUNK), jnp.int32),
            pltpu.VMEM((nchunk, _CHUNK), jnp.int32),
            pltpu.VMEM((_NBUF, _CHUNK, E), jnp.float32),
            pltpu.VMEM((_TBUF, eb, 8, _CHUNK + 1), jnp.float32),
            pltpu.SemaphoreType.DMA((_NBUF,)),
            pltpu.SemaphoreType.DMA((_TBUF,)),
        ],
    )
    def gather_kernel(idx_hbm, table_hbm, out_hbm, idx_v, u_v, rows_v, tr_v,
                      gsem, osem):
        wid = lax.axis_index("s") * nc + lax.axis_index("c")
        c0 = wid * nchunk
        pltpu.sync_copy(idx_hbm.at[wid], idx_v)

        b0 = _b0(V)

        @plsc.parallel_loop(0, nchunk * (_CHUNK // 16), unroll=4)
        def _(t):
            c = t >> 3
            g = t & 7
            iv = idx_v[c, pl.ds(g * 16, 16)]
            hv = iv >> _HALF_BITS
            u_v[c, pl.ds(g * 16, 16)] = ((iv - hv * b0) << 1) | hv

        zeros16 = jnp.zeros((16,), jnp.int32)
        iota = lax.iota(jnp.int32, 16)
        eb_vecs = [(16 * k + iota) // 8 for k in range(E // 16)]
        es_vecs = [(16 * k + iota) % 8 for k in range(E // 16)]

        def gather_desc(j, b):
            return pltpu.make_async_copy(
                table_hbm.at[u_v.at[j]], rows_v.at[b], gsem.at[b])

        def ostore_desc(j, b):
            c = c0 + j
            s_ = c >> 7
            nb = c & (nb_per_s - 1)
            return pltpu.make_async_copy(
                tr_v.at[b, :, :, pl.ds(0, _CHUNK)],
                out_hbm.at[s_, :, nb], osem.at[b])

        for b in range(_NBUF):
            gather_desc(b, b).start()

        def round_body(r, carry):
            j0 = r * _NBUF
            for b in range(_NBUF):
                j = j0 + b
                gather_desc(j, b).wait()

                @pl.when(r > 0)
                def _():
                    ostore_desc(j - _NBUF, b).wait()

                rows = rows_v.at[b]
                tr = tr_v.at[b]

                @plsc.parallel_loop(0, _CHUNK, unroll=2)
                def _(n):
                    col = zeros16 + n
                    for k in range(E // 16):
                        v = rows[n, pl.ds(16 * k, 16)]
                        plsc.store_scatter(tr, [eb_vecs[k], es_vecs[k], col], v)

                ostore_desc(j, b).start()

                @pl.when(j + _NBUF < nchunk)
                def _():
                    gather_desc(j + _NBUF, b).start()
            return carry

        lax.fori_loop(0, nchunk // _NBUF, round_body, 0)

        j0 = nchunk - _NBUF
        for b in range(_NBUF):
            ostore_desc(j0 + b, b).wait()

    return gather_kernel, nw, nchunk


@functools.lru_cache(maxsize=None)
def _make_relayout(V, E):
    assert _HALF % _VBLK == 0 and E % 8 == 0
    grid = _HALF // _VBLK
    b0_blk = _b0(V) // _VBLK

    def body(a_ref, b_ref, out_ref):
        out_ref[:, 0:E] = a_ref[...].T
        out_ref[:, E:2 * E] = b_ref[...].T

    return pl.pallas_call(
        body,
        grid=(grid,),
        in_specs=[
            pl.BlockSpec((E, _VBLK), lambda i: (0, i)),
            pl.BlockSpec((E, _VBLK), lambda i: (0, i + b0_blk)),
        ],
        out_specs=pl.BlockSpec((_VBLK, 2 * E), lambda i: (i, 0)),
        out_shape=jax.ShapeDtypeStruct((_HALF, 2 * E), jnp.float32),
    )


def kernel(x, table):
    n, s = x.shape
    v, e = table.shape
    assert _HALF < v <= 2 * _HALF
    fn, nw, nchunk = _make_gather(s, n, v, e)
    idx = x.astype(jnp.int32).T.reshape(nw, nchunk, _CHUNK)
    tt = table.T
    pairs = _make_relayout(v, e)(tt, tt)
    table_lin = pairs.reshape(2 * _HALF, e)
    k6 = fn(idx, table_lin)
    return k6.transpose((2, 4, 0, 1, 3)).reshape(n, s, e)


# speedup vs baseline: 4.5973x; 1.2590x over previous
import functools

import jax
import jax.numpy as jnp
from jax import lax
from jax.experimental import pallas as pl
from jax.experimental.pallas import tpu as pltpu
from jax.experimental.pallas import tpu_sc as plsc

_CHUNK = 128
_NBUF = 4
_TBUF = 4
_HALF_BITS = 19
_HALF = 1 << _HALF_BITS
_VBLK = 8192


def _b0(V):
    return (V - _HALF + _VBLK - 1) // _VBLK * _VBLK


@functools.lru_cache(maxsize=None)
def _make_gather(S, N, V, E):
    info = plsc.get_sparse_core_info()
    nc, ns = info.num_cores, info.num_subcores
    nw = nc * ns
    B = S * N
    assert B % (nw * _CHUNK) == 0 and N % _CHUNK == 0 and E % 8 == 0
    nchunk = B // (nw * _CHUNK)
    nb_per_s = N // _CHUNK
    eb = E // 8
    assert nchunk % _NBUF == 0 and nchunk > _NBUF
    mesh = plsc.VectorSubcoreMesh(core_axis_name="c", subcore_axis_name="s")

    @functools.partial(
        pl.kernel,
        out_type=jax.ShapeDtypeStruct((S, eb, nb_per_s, 8, _CHUNK), jnp.float32),
        mesh=mesh,
        compiler_params=pltpu.CompilerParams(
            use_tc_tiling_on_sc=False, needs_layout_passes=False),
        scratch_types=[
            pltpu.VMEM((nchunk, _CHUNK), jnp.int32),
            pltpu.VMEM((nchunk, _CHUNK), jnp.int32),
            pltpu.VMEM((_NBUF, _CHUNK, E), jnp.float32),
            pltpu.VMEM((_TBUF, eb, 8, _CHUNK + 1), jnp.float32),
            pltpu.SemaphoreType.DMA((_NBUF,)),
            pltpu.SemaphoreType.DMA((_TBUF,)),
        ],
    )
    def gather_kernel(idx_hbm, table_hbm, out_hbm, idx_v, u_v, rows_v, tr_v,
                      gsem, osem):
        wid = lax.axis_index("s") * nc + lax.axis_index("c")
        c0 = wid * nchunk
        pltpu.sync_copy(idx_hbm.at[wid], idx_v)

        b0 = _b0(V)

        @plsc.parallel_loop(0, nchunk * (_CHUNK // 16), unroll=4)
        def _(t):
            c = t >> 3
            g = t & 7
            iv = idx_v[c, pl.ds(g * 16, 16)]
            hv = iv >> _HALF_BITS
            u_v[c, pl.ds(g * 16, 16)] = ((iv - hv * b0) << 1) | hv

        zeros16 = jnp.zeros((16,), jnp.int32)
        iota = lax.iota(jnp.int32, 16)
        eb_vecs = [(16 * k + iota) // 8 for k in range(E // 16)]
        es_vecs = [(16 * k + iota) % 8 for k in range(E // 16)]

        def gather_desc(j, b):
            return pltpu.make_async_copy(
                table_hbm.at[u_v.at[j]], rows_v.at[b], gsem.at[b])

        def ostore_desc(j, b):
            c = c0 + j
            s_ = c >> 7
            nb = c & (nb_per_s - 1)
            return pltpu.make_async_copy(
                tr_v.at[b, :, :, pl.ds(0, _CHUNK)],
                out_hbm.at[s_, :, nb], osem.at[b])

        for b in range(_NBUF):
            gather_desc(b, b).start()

        def round_body(r, carry):
            j0 = r * _NBUF
            for b in range(_NBUF):
                j = j0 + b
                gather_desc(j, b).wait()

                @pl.when(r > 0)
                def _():
                    ostore_desc(j - _NBUF, b).wait()

                rows = rows_v.at[b]
                tr = tr_v.at[b]

                @plsc.parallel_loop(0, _CHUNK, unroll=2)
                def _(n):
                    col = zeros16 + n
                    for k in range(E // 16):
                        v = rows[n, pl.ds(16 * k, 16)]
                        plsc.store_scatter(tr, [eb_vecs[k], es_vecs[k], col], v)

                ostore_desc(j, b).start()

                @pl.when(j + _NBUF < nchunk)
                def _():
                    gather_desc(j + _NBUF, b).start()
            return carry

        lax.fori_loop(0, nchunk // _NBUF, round_body, 0)

        j0 = nchunk - _NBUF
        for b in range(_NBUF):
            ostore_desc(j0 + b, b).wait()

    return gather_kernel, nw, nchunk


@functools.lru_cache(maxsize=None)
def _make_relayout(V, E):
    assert _HALF % _VBLK == 0 and E % 8 == 0
    grid = _HALF // _VBLK
    b0_blk = _b0(V) // _VBLK

    def body(a_ref, b_ref, out_ref):
        out_ref[:, 0:E] = a_ref[...].T
        out_ref[:, E:2 * E] = b_ref[...].T

    return pl.pallas_call(
        body,
        grid=(grid,),
        in_specs=[
            pl.BlockSpec((E, _VBLK), lambda i: (0, i)),
            pl.BlockSpec((E, _VBLK), lambda i: (0, i + b0_blk)),
        ],
        out_specs=pl.BlockSpec((_VBLK, 2 * E), lambda i: (i, 0)),
        out_shape=jax.ShapeDtypeStruct((_HALF, 2 * E), jnp.float32),
    )


def kernel(x, table):
    n, s = x.shape
    v, e = table.shape
    assert _HALF < v <= 2 * _HALF
    fn, nw, nchunk = _make_gather(s, n, v, e)
    idx = x.astype(jnp.int32).T.reshape(nw, nchunk, _CHUNK)
    tt = table.T
    pairs = _make_relayout(v, e)(tt, tt)
    table_lin = pairs.reshape(2 * _HALF, e)
    k6 = fn(idx, table_lin)
    return k6.transpose((2, 4, 0, 1, 3)).reshape(n, s, e)


# speedup vs baseline: 4.7691x; 1.0374x over previous
import functools

import jax
import jax.numpy as jnp
from jax import lax
from jax.experimental import pallas as pl
from jax.experimental.pallas import tpu as pltpu
from jax.experimental.pallas import tpu_sc as plsc

_CHUNK = 128
_NBUF = 4
_TBUF = 4
_HALF_BITS = 19
_HALF = 1 << _HALF_BITS
_VBLK = 16384


def _b0(V):
    return (V - _HALF + _VBLK - 1) // _VBLK * _VBLK


@functools.lru_cache(maxsize=None)
def _make_gather(S, N, V, E):
    info = plsc.get_sparse_core_info()
    nc, ns = info.num_cores, info.num_subcores
    nw = nc * ns
    B = S * N
    assert B % (nw * _CHUNK) == 0 and N % _CHUNK == 0 and E % 8 == 0
    nchunk = B // (nw * _CHUNK)
    nb_per_s = N // _CHUNK
    eb = E // 8
    assert nchunk % _NBUF == 0 and nchunk > _NBUF
    mesh = plsc.VectorSubcoreMesh(core_axis_name="c", subcore_axis_name="s")

    @functools.partial(
        pl.kernel,
        out_type=jax.ShapeDtypeStruct((S, eb, nb_per_s, 8, _CHUNK), jnp.float32),
        mesh=mesh,
        compiler_params=pltpu.CompilerParams(
            use_tc_tiling_on_sc=False, needs_layout_passes=False),
        scratch_types=[
            pltpu.VMEM((nchunk, _CHUNK), jnp.int32),
            pltpu.VMEM((nchunk, _CHUNK), jnp.int32),
            pltpu.VMEM((_NBUF, _CHUNK, E), jnp.float32),
            pltpu.VMEM((_TBUF, eb, 8, _CHUNK + 1), jnp.float32),
            pltpu.SemaphoreType.DMA((_NBUF,)),
            pltpu.SemaphoreType.DMA((_TBUF,)),
        ],
    )
    def gather_kernel(idx_hbm, table_hbm, out_hbm, idx_v, u_v, rows_v, tr_v,
                      gsem, osem):
        wid = lax.axis_index("s") * nc + lax.axis_index("c")
        c0 = wid * nchunk
        pltpu.sync_copy(idx_hbm.at[wid], idx_v)

        b0 = _b0(V)

        @plsc.parallel_loop(0, nchunk * (_CHUNK // 16), unroll=4)
        def _(t):
            c = t >> 3
            g = t & 7
            iv = idx_v[c, pl.ds(g * 16, 16)]
            hv = iv >> _HALF_BITS
            u_v[c, pl.ds(g * 16, 16)] = ((iv - hv * b0) << 1) | hv

        zeros16 = jnp.zeros((16,), jnp.int32)
        iota = lax.iota(jnp.int32, 16)
        eb_vecs = [(16 * k + iota) // 8 for k in range(E // 16)]
        es_vecs = [(16 * k + iota) % 8 for k in range(E // 16)]

        def gather_desc(j, b):
            return pltpu.make_async_copy(
                table_hbm.at[u_v.at[j]], rows_v.at[b], gsem.at[b])

        def ostore_desc(j, b):
            c = c0 + j
            s_ = c >> 7
            nb = c & (nb_per_s - 1)
            return pltpu.make_async_copy(
                tr_v.at[b, :, :, pl.ds(0, _CHUNK)],
                out_hbm.at[s_, :, nb], osem.at[b])

        for b in range(_NBUF):
            gather_desc(b, b).start()

        def round_body(r, carry):
            j0 = r * _NBUF
            for b in range(_NBUF):
                j = j0 + b
                gather_desc(j, b).wait()

                @pl.when(r > 0)
                def _():
                    ostore_desc(j - _NBUF, b).wait()

                rows = rows_v.at[b]
                tr = tr_v.at[b]

                @plsc.parallel_loop(0, _CHUNK, unroll=2)
                def _(n):
                    col = zeros16 + n
                    for k in range(E // 16):
                        v = rows[n, pl.ds(16 * k, 16)]
                        plsc.store_scatter(tr, [eb_vecs[k], es_vecs[k], col], v)

                ostore_desc(j, b).start()

                @pl.when(j + _NBUF < nchunk)
                def _():
                    gather_desc(j + _NBUF, b).start()
            return carry

        lax.fori_loop(0, nchunk // _NBUF, round_body, 0)

        j0 = nchunk - _NBUF
        for b in range(_NBUF):
            ostore_desc(j0 + b, b).wait()

    return gather_kernel, nw, nchunk


@functools.lru_cache(maxsize=None)
def _make_relayout(V, E):
    assert _HALF % _VBLK == 0 and E % 8 == 0
    grid = _HALF // _VBLK
    b0_blk = _b0(V) // _VBLK

    def body(a_ref, b_ref, out_ref):
        out_ref[:, 0:E] = a_ref[...].T
        out_ref[:, E:2 * E] = b_ref[...].T

    return pl.pallas_call(
        body,
        grid=(grid,),
        in_specs=[
            pl.BlockSpec((E, _VBLK), lambda i: (0, i)),
            pl.BlockSpec((E, _VBLK), lambda i: (0, i + b0_blk)),
        ],
        out_specs=pl.BlockSpec((_VBLK, 2 * E), lambda i: (i, 0)),
        out_shape=jax.ShapeDtypeStruct((_HALF, 2 * E), jnp.float32),
    )


def kernel(x, table):
    n, s = x.shape
    v, e = table.shape
    assert _HALF < v <= 2 * _HALF
    fn, nw, nchunk = _make_gather(s, n, v, e)
    idx = x.astype(jnp.int32).T.reshape(nw, nchunk, _CHUNK)
    tt = table.T
    pairs = _make_relayout(v, e)(tt, tt)
    table_lin = pairs.reshape(2 * _HALF, e)
    k6 = fn(idx, table_lin)
    return k6.transpose((2, 4, 0, 1, 3)).reshape(n, s, e)


# speedup vs baseline: 4.7958x; 1.0056x over previous
import functools

import jax
import jax.numpy as jnp
from jax import lax
from jax.experimental import pallas as pl
from jax.experimental.pallas import tpu as pltpu
from jax.experimental.pallas import tpu_sc as plsc

_CHUNK = 128
_NBUF = 5
_TBUF = 5
_HALF_BITS = 19
_HALF = 1 << _HALF_BITS
_VBLK = 16384


def _b0(V):
    return (V - _HALF + _VBLK - 1) // _VBLK * _VBLK


@functools.lru_cache(maxsize=None)
def _make_gather(S, N, V, E):
    info = plsc.get_sparse_core_info()
    nc, ns = info.num_cores, info.num_subcores
    nw = nc * ns
    B = S * N
    assert B % (nw * _CHUNK) == 0 and N % _CHUNK == 0 and E % 8 == 0
    nchunk = B // (nw * _CHUNK)
    nb_per_s = N // _CHUNK
    eb = E // 8
    assert nchunk % _NBUF == 0 and nchunk > _NBUF
    mesh = plsc.VectorSubcoreMesh(core_axis_name="c", subcore_axis_name="s")

    @functools.partial(
        pl.kernel,
        out_type=jax.ShapeDtypeStruct((S, eb, nb_per_s, 8, _CHUNK), jnp.float32),
        mesh=mesh,
        compiler_params=pltpu.CompilerParams(
            use_tc_tiling_on_sc=False, needs_layout_passes=False),
        scratch_types=[
            pltpu.VMEM((nchunk, _CHUNK), jnp.int32),
            pltpu.VMEM((_NBUF, _CHUNK, E), jnp.float32),
            pltpu.VMEM((_TBUF, eb, 8, _CHUNK + 1), jnp.float32),
            pltpu.SemaphoreType.DMA((_NBUF,)),
            pltpu.SemaphoreType.DMA((_TBUF,)),
        ],
    )
    def gather_kernel(idx_hbm, table_hbm, out_hbm, idx_v, rows_v, tr_v,
                      gsem, osem):
        wid = lax.axis_index("s") * nc + lax.axis_index("c")
        c0 = wid * nchunk
        pltpu.sync_copy(idx_hbm.at[wid], idx_v)

        b0 = _b0(V)

        @plsc.parallel_loop(0, nchunk * (_CHUNK // 16), unroll=4)
        def _(t):
            c = t >> 3
            g = t & 7
            iv = idx_v[c, pl.ds(g * 16, 16)]
            hv = iv >> _HALF_BITS
            idx_v[c, pl.ds(g * 16, 16)] = ((iv - hv * b0) << 1) | hv

        zeros16 = jnp.zeros((16,), jnp.int32)
        iota = lax.iota(jnp.int32, 16)
        eb_vecs = [(16 * k + iota) // 8 for k in range(E // 16)]
        es_vecs = [(16 * k + iota) % 8 for k in range(E // 16)]

        def gather_desc(j, b):
            return pltpu.make_async_copy(
                table_hbm.at[idx_v.at[j]], rows_v.at[b], gsem.at[b])

        def ostore_desc(j, b):
            c = c0 + j
            s_ = c >> 7
            nb = c & (nb_per_s - 1)
            return pltpu.make_async_copy(
                tr_v.at[b, :, :, pl.ds(0, _CHUNK)],
                out_hbm.at[s_, :, nb], osem.at[b])

        for b in range(_NBUF):
            gather_desc(b, b).start()

        def round_body(r, carry):
            j0 = r * _NBUF
            for b in range(_NBUF):
                j = j0 + b
                gather_desc(j, b).wait()

                @pl.when(r > 0)
                def _():
                    ostore_desc(j - _NBUF, b).wait()

                rows = rows_v.at[b]
                tr = tr_v.at[b]

                @plsc.parallel_loop(0, _CHUNK, unroll=2)
                def _(n):
                    col = zeros16 + n
                    for k in range(E // 16):
                        v = rows[n, pl.ds(16 * k, 16)]
                        plsc.store_scatter(tr, [eb_vecs[k], es_vecs[k], col], v)

                ostore_desc(j, b).start()

                @pl.when(j + _NBUF < nchunk)
                def _():
                    gather_desc(j + _NBUF, b).start()
            return carry

        lax.fori_loop(0, nchunk // _NBUF, round_body, 0)

        j0 = nchunk - _NBUF
        for b in range(_NBUF):
            ostore_desc(j0 + b, b).wait()

    return gather_kernel, nw, nchunk


@functools.lru_cache(maxsize=None)
def _make_relayout(V, E):
    assert _HALF % _VBLK == 0 and E % 8 == 0
    grid = _HALF // _VBLK
    b0_blk = _b0(V) // _VBLK

    def body(a_ref, b_ref, out_ref):
        out_ref[:, 0:E] = a_ref[...].T
        out_ref[:, E:2 * E] = b_ref[...].T

    return pl.pallas_call(
        body,
        grid=(grid,),
        in_specs=[
            pl.BlockSpec((E, _VBLK), lambda i: (0, i)),
            pl.BlockSpec((E, _VBLK), lambda i: (0, i + b0_blk)),
        ],
        out_specs=pl.BlockSpec((_VBLK, 2 * E), lambda i: (i, 0)),
        out_shape=jax.ShapeDtypeStruct((_HALF, 2 * E), jnp.float32),
    )


def kernel(x, table):
    n, s = x.shape
    v, e = table.shape
    assert _HALF < v <= 2 * _HALF
    fn, nw, nchunk = _make_gather(s, n, v, e)
    idx = x.astype(jnp.int32).T.reshape(nw, nchunk, _CHUNK)
    tt = table.T
    pairs = _make_relayout(v, e)(tt, tt)
    table_lin = pairs.reshape(2 * _HALF, e)
    k6 = fn(idx, table_lin)
    return k6.transpose((2, 4, 0, 1, 3)).reshape(n, s, e)
